# Initial kernel scaffold; baseline (speedup 1.0000x reference)
import jax
import jax.numpy as jnp
from jax.experimental import pallas as pl


def kernel(features, edge_index, W1, attn_l1, attn_r1, b1, W2, attn_l2, attn_r2, b2):
    raise NotImplementedError("write your pallas kernel here")


# speedup vs baseline: 41.1940x; 41.1940x over previous
import functools

import jax
import jax.numpy as jnp
from jax import lax
from jax.experimental import pallas as pl
from jax.experimental.pallas import tpu as pltpu
from jax.experimental.pallas import tpu_sc as plsc

_N = 10000
_E = 320000
_H = 8
_F = 16
_D1 = 144
_D2 = 16
_NC = 2
_NS = 16
_NTILES = _NC * _NS
_EPT = _E // _NTILES
_CHUNK = 80
_NCH = _EPT // _CHUNK
_RPS = 624
_RTAIL = _N - _NS * _RPS


def _tc1_body(x_ref, w_ref, al_ref, ar_ref, zext_ref, l_ref, r_ref):
    z = jnp.dot(x_ref[...], w_ref[...], preferred_element_type=jnp.float32)
    el = jnp.dot(z, al_ref[...], preferred_element_type=jnp.float32)
    er = jnp.dot(z, ar_ref[...], preferred_element_type=jnp.float32)
    n = z.shape[0]
    zext_ref[...] = jnp.concatenate(
        [z, jnp.ones((n, _H), jnp.float32), jnp.zeros((n, _H), jnp.float32)],
        axis=1)
    l_ref[...] = jnp.concatenate([el, el], axis=1)
    r_ref[...] = jnp.concatenate([er, er], axis=1)


def _tc1(x, W1, AL, AR):
    return pl.pallas_call(
        _tc1_body,
        out_shape=(
            jax.ShapeDtypeStruct((_N, _D1), jnp.float32),
            jax.ShapeDtypeStruct((_N, 16), jnp.float32),
            jax.ShapeDtypeStruct((_N, 16), jnp.float32),
        ),
    )(x, W1, AL, AR)


def _sc_edge_pass(src, dst, zext, ltab, rtab, d, multihead):
    mesh = plsc.VectorSubcoreMesh(core_axis_name="c", subcore_axis_name="s")
    zeros = jnp.zeros((_N, d), jnp.float32)

    @functools.partial(
        pl.kernel,
        mesh=mesh,
        out_type=jax.ShapeDtypeStruct((_NC, _N, d), jnp.float32),
        compiler_params=pltpu.CompilerParams(use_tc_tiling_on_sc=False),
        scratch_types=[
            pltpu.VMEM_SHARED((_N, d), jnp.float32),
            pltpu.VMEM((_CHUNK,), jnp.int32),
            pltpu.VMEM((_CHUNK,), jnp.int32),
            pltpu.VMEM((_CHUNK, 16), jnp.float32),
            pltpu.VMEM((_CHUNK, 16), jnp.float32),
            pltpu.VMEM((_CHUNK, 16), jnp.float32),
            pltpu.VMEM((_CHUNK, d), jnp.float32),
            pltpu.SemaphoreType.DMA,
        ],
    )
    def k(src_hbm, dst_hbm, z_hbm, l_hbm, r_hbm, zero_hbm, out_hbm,
          acc, srcv, dstv, lv, rv, wv, zv, sem):
        cid = lax.axis_index("c")
        sid = lax.axis_index("s")
        wid = cid * _NS + sid
        pltpu.sync_copy(zero_hbm.at[pl.ds(sid * _RPS, _RPS)],
                        acc.at[pl.ds(sid * _RPS, _RPS)])

        @pl.when(sid == _NS - 1)
        def _zero_tail():
            pltpu.sync_copy(zero_hbm.at[pl.ds(_NS * _RPS, _RTAIL)],
                            acc.at[pl.ds(_NS * _RPS, _RTAIL)])

        plsc.subcore_barrier()

        base = wid * _EPT

        @pl.loop(0, _NCH)
        def _chunk(g):
            off = base + g * _CHUNK
            pltpu.sync_copy(src_hbm.at[pl.ds(off, _CHUNK)], srcv)
            pltpu.sync_copy(dst_hbm.at[pl.ds(off, _CHUNK)], dstv)
            pltpu.async_copy(l_hbm.at[srcv], lv, sem).wait()
            pltpu.async_copy(r_hbm.at[dstv], rv, sem).wait()
            pltpu.async_copy(z_hbm.at[srcv], zv, sem).wait()

            if multihead:
                @pl.loop(0, _CHUNK)
                def _w(ci):
                    e = lv[ci] + rv[ci]
                    e = jnp.where(e >= 0.0, e, 0.2 * e)
                    wv[ci] = jnp.exp(e)

                @pl.loop(0, _CHUNK)
                def _msg(ci):
                    wrow = wv[ci]
                    for h in range(_H):
                        ws = wrow[h]
                        zv[ci, pl.ds(h * _F, 16)] = ws * zv[ci, pl.ds(h * _F, 16)]
                    zv[ci, pl.ds(_H * _F, 16)] = wv[ci] * zv[ci, pl.ds(_H * _F, 16)]
            else:
                @pl.loop(0, _CHUNK)
                def _w(ci):
                    e = lv[ci] + rv[ci]
                    e = jnp.where(e >= 0.0, e, 0.2 * e)
                    zv[ci] = jnp.exp(e) * zv[ci]

            pltpu.sync_copy(zv, acc.at[dstv], add=True)

        plsc.subcore_barrier()
        pltpu.sync_copy(acc.at[pl.ds(sid * _RPS, _RPS)],
                        out_hbm.at[cid, pl.ds(sid * _RPS, _RPS)])

        @pl.when(sid == _NS - 1)
        def _out_tail():
            pltpu.sync_copy(acc.at[pl.ds(_NS * _RPS, _RTAIL)],
                            out_hbm.at[cid, pl.ds(_NS * _RPS, _RTAIL)])

    return k(src, dst, zext, ltab, rtab, zeros)


def _tc2_body(accp_ref, b1_ref, w2_ref, va_ref, vb_ref, expand_ref,
              zext_ref, l_ref, r_ref):
    acc = accp_ref[0] + accp_ref[1]
    s = acc[:, _H * _F:_H * _F + _H]
    sx = jnp.dot(s, expand_ref[...], preferred_element_type=jnp.float32)
    h = acc[:, 0:_H * _F] / (sx + 1e-9) + b1_ref[...]
    h = jnp.where(h > 0.0, h, jnp.exp(h) - 1.0)
    z2 = jnp.dot(h, w2_ref[...], preferred_element_type=jnp.float32)
    el2 = jnp.dot(h, va_ref[...], preferred_element_type=jnp.float32)
    er2 = jnp.dot(h, vb_ref[...], preferred_element_type=jnp.float32)
    n = h.shape[0]
    one16 = jnp.ones((1, 16), jnp.float32)
    zext_ref[...] = jnp.concatenate(
        [z2, jnp.ones((n, 1), jnp.float32), jnp.zeros((n, 12), jnp.float32)],
        axis=1)
    l_ref[...] = jnp.dot(el2, one16, preferred_element_type=jnp.float32)
    r_ref[...] = jnp.dot(er2, one16, preferred_element_type=jnp.float32)


def _tc2(accp, b1, W2, va, vb, EXPAND):
    return pl.pallas_call(
        _tc2_body,
        out_shape=(
            jax.ShapeDtypeStruct((_N, _D2), jnp.float32),
            jax.ShapeDtypeStruct((_N, 16), jnp.float32),
            jax.ShapeDtypeStruct((_N, 16), jnp.float32),
        ),
    )(accp, b1, W2, va, vb, EXPAND)


def _tc3_body(accp_ref, b2_ref, out_ref):
    acc = accp_ref[0] + accp_ref[1]
    sb = jnp.dot(acc[:, 3:4], jnp.ones((1, 16), jnp.float32),
                 preferred_element_type=jnp.float32)
    out_ref[...] = acc[:, 0:3] / (sb[:, 0:3] + 1e-9) + b2_ref[...]


def _tc3(accp, b2):
    return pl.pallas_call(
        _tc3_body,
        out_shape=jax.ShapeDtypeStruct((_N, 3), jnp.float32),
    )(accp, b2)


def kernel(features, edge_index, W1, attn_l1, attn_r1, b1,
           W2, attn_l2, attn_r2, b2):
    src = edge_index[0]
    dst = edge_index[1]
    eye8 = jnp.eye(_H, dtype=jnp.float32)
    AL = (attn_l1[:, :, None] * eye8[:, None, :]).reshape(_H * _F, _H)
    AR = (attn_r1[:, :, None] * eye8[:, None, :]).reshape(_H * _F, _H)
    EXPAND = jnp.kron(eye8, jnp.ones((1, _F), jnp.float32))
    va = (W2 @ attn_l2[0]).reshape(_H * _F, 1)
    vb = (W2 @ attn_r2[0]).reshape(_H * _F, 1)

    zext1, L1, R1 = _tc1(features, W1, AL, AR)
    acc1 = _sc_edge_pass(src, dst, zext1, L1, R1, _D1, True)
    zext2, L2, R2 = _tc2(acc1, b1.reshape(1, _H * _F), W2, va, vb, EXPAND)
    acc2 = _sc_edge_pass(src, dst, zext2, L2, R2, _D2, False)
    out = _tc3(acc2, b2.reshape(1, 3))
    return out.reshape(_N, 1, 3)


# speedup vs baseline: 72.2522x; 1.7540x over previous
import functools

import jax
import jax.numpy as jnp
from jax import lax
from jax.experimental import pallas as pl
from jax.experimental.pallas import tpu as pltpu
from jax.experimental.pallas import tpu_sc as plsc

_N = 10000
_E = 320000
_H = 8
_F = 16
_D1 = 144
_D2 = 16
_NC = 2
_NS = 16
_NTILES = _NC * _NS
_EPT = _E // _NTILES
_CHUNK = 50
_NCH = _EPT // _CHUNK
_RPS = 624
_RTAIL = _N - _NS * _RPS


def _tc1_body(x_ref, w_ref, al_ref, ar_ref, zext_ref, r_ref):
    z = jnp.dot(x_ref[...], w_ref[...], preferred_element_type=jnp.float32)
    el = jnp.dot(z, al_ref[...], preferred_element_type=jnp.float32)
    er = jnp.dot(z, ar_ref[...], preferred_element_type=jnp.float32)
    n = z.shape[0]
    zext_ref[...] = jnp.concatenate(
        [z, jnp.ones((n, _H), jnp.float32), el], axis=1)
    r_ref[...] = jnp.concatenate([er, er], axis=1)


def _tc1(x, W1, AL, AR):
    return pl.pallas_call(
        _tc1_body,
        out_shape=(
            jax.ShapeDtypeStruct((_N, _D1), jnp.float32),
            jax.ShapeDtypeStruct((_N, 16), jnp.float32),
        ),
    )(x, W1, AL, AR)


def _sc_edge_pass(src3, dst3, zext, rtab, d, multihead):
    mesh = plsc.VectorSubcoreMesh(core_axis_name="c", subcore_axis_name="s")
    zeros = jnp.zeros((_N, d), jnp.float32)

    @functools.partial(
        pl.kernel,
        mesh=mesh,
        out_type=jax.ShapeDtypeStruct((_NC, _N, d), jnp.float32),
        compiler_params=pltpu.CompilerParams(use_tc_tiling_on_sc=False,
                                             needs_layout_passes=False),
        scratch_types=[
            pltpu.VMEM_SHARED((_N, d), jnp.float32),
            pltpu.VMEM((_NCH, _CHUNK), jnp.int32),
            pltpu.VMEM((_NCH, _CHUNK), jnp.int32),
            pltpu.VMEM((_CHUNK, 16), jnp.float32),
            pltpu.VMEM((_CHUNK, 16), jnp.float32),
            pltpu.VMEM((_CHUNK, d), jnp.float32),
            pltpu.VMEM((_CHUNK, d), jnp.float32),
            pltpu.SemaphoreType.DMA,
            pltpu.SemaphoreType.DMA,
            pltpu.SemaphoreType.DMA,
            pltpu.SemaphoreType.DMA,
        ],
    )
    def k(src_hbm, dst_hbm, z_hbm, r_hbm, zero_hbm, out_hbm,
          acc, srcv, dstv, rv0, rv1, zv0, zv1, sg0, sg1, ss0, ss1):
        cid = lax.axis_index("c")
        sid = lax.axis_index("s")
        wid = cid * _NS + sid
        rv = (rv0, rv1)
        zv = (zv0, zv1)
        sg = (sg0, sg1)
        ss = (ss0, ss1)

        pltpu.sync_copy(zero_hbm.at[pl.ds(sid * _RPS, _RPS)],
                        acc.at[pl.ds(sid * _RPS, _RPS)])

        @pl.when(sid == _NS - 1)
        def _zero_tail():
            pltpu.sync_copy(zero_hbm.at[pl.ds(_NS * _RPS, _RTAIL)],
                            acc.at[pl.ds(_NS * _RPS, _RTAIL)])

        pltpu.sync_copy(src_hbm.at[wid], srcv)
        pltpu.sync_copy(dst_hbm.at[wid], dstv)
        plsc.subcore_barrier()

        def issue_gather(b, g):
            pltpu.async_copy(r_hbm.at[dstv.at[g]], rv[b], sg[b])
            pltpu.async_copy(z_hbm.at[srcv.at[g]], zv[b], sg[b])

        def wait_gather(b, g):
            pltpu.make_async_copy(r_hbm.at[dstv.at[g]], rv[b], sg[b]).wait()
            pltpu.make_async_copy(z_hbm.at[srcv.at[g]], zv[b], sg[b]).wait()

        def issue_scatter(b, g):
            pltpu.async_copy(zv[b], acc.at[dstv.at[g]], ss[b], add=True)

        def wait_scatter(b, g):
            pltpu.make_async_copy(zv[b], acc.at[dstv.at[g]], ss[b]).wait()

        def compute(b):
            z = zv[b]
            r = rv[b]
            if multihead:
                @pl.loop(0, _CHUNK)
                def _msg(ci):
                    tail = z[ci, pl.ds(_H * _F, 16)]
                    e = tail + r[ci]
                    e = jnp.where(e >= 0.0, e, 0.2 * e)
                    w16 = jnp.exp(e)
                    for h in range(_H):
                        ws = w16[8 + h]
                        z[ci, pl.ds(h * _F, 16)] = ws * z[ci, pl.ds(h * _F, 16)]
                    z[ci, pl.ds(_H * _F, 16)] = lax.rev(w16, (0,)) * tail
            else:
                @pl.loop(0, _CHUNK)
                def _msg(ci):
                    zrow = z[ci]
                    bc = plsc.load_gather(
                        z, [jnp.full((16,), ci, jnp.int32),
                            jnp.full((16,), 4, jnp.int32)])
                    e = bc + r[ci]
                    e = jnp.where(e >= 0.0, e, 0.2 * e)
                    z[ci] = jnp.exp(e) * zrow

        issue_gather(0, 0)

        @pl.loop(0, _NCH, step=2)
        def _pair(g):
            issue_gather(1, g + 1)
            wait_gather(0, g)
            compute(0)
            issue_scatter(0, g)
            wait_gather(1, g + 1)
            compute(1)
            issue_scatter(1, g + 1)
            wait_scatter(0, g)

            @pl.when(g + 2 < _NCH)
            def _next():
                issue_gather(0, g + 2)

            wait_scatter(1, g + 1)

        plsc.subcore_barrier()
        pltpu.sync_copy(acc.at[pl.ds(sid * _RPS, _RPS)],
                        out_hbm.at[cid, pl.ds(sid * _RPS, _RPS)])

        @pl.when(sid == _NS - 1)
        def _out_tail():
            pltpu.sync_copy(acc.at[pl.ds(_NS * _RPS, _RTAIL)],
                            out_hbm.at[cid, pl.ds(_NS * _RPS, _RTAIL)])

    return k(src3, dst3, zext, rtab, zeros)


def _tc2_body(accp_ref, b1_ref, w2_ref, va_ref, vb_ref, expand_ref,
              zext_ref, r_ref):
    acc = accp_ref[0] + accp_ref[1]
    s = acc[:, _H * _F:_H * _F + _H]
    sx = jnp.dot(s, expand_ref[...], preferred_element_type=jnp.float32)
    h = acc[:, 0:_H * _F] / (sx + 1e-9) + b1_ref[...]
    h = jnp.where(h > 0.0, h, jnp.exp(h) - 1.0)
    z2 = jnp.dot(h, w2_ref[...], preferred_element_type=jnp.float32)
    el2 = jnp.dot(h, va_ref[...], preferred_element_type=jnp.float32)
    er2 = jnp.dot(h, vb_ref[...], preferred_element_type=jnp.float32)
    n = h.shape[0]
    one16 = jnp.ones((1, 16), jnp.float32)
    zext_ref[...] = jnp.concatenate(
        [z2, jnp.ones((n, 1), jnp.float32), el2,
         jnp.zeros((n, 11), jnp.float32)], axis=1)
    r_ref[...] = jnp.dot(er2, one16, preferred_element_type=jnp.float32)


def _tc2(accp, b1, W2, va, vb, EXPAND):
    return pl.pallas_call(
        _tc2_body,
        out_shape=(
            jax.ShapeDtypeStruct((_N, _D2), jnp.float32),
            jax.ShapeDtypeStruct((_N, 16), jnp.float32),
        ),
    )(accp, b1, W2, va, vb, EXPAND)


def _tc3_body(accp_ref, b2_ref, out_ref):
    acc = accp_ref[0] + accp_ref[1]
    sb = jnp.dot(acc[:, 3:4], jnp.ones((1, 16), jnp.float32),
                 preferred_element_type=jnp.float32)
    out_ref[...] = acc[:, 0:3] / (sb[:, 0:3] + 1e-9) + b2_ref[...]


def _tc3(accp, b2):
    return pl.pallas_call(
        _tc3_body,
        out_shape=jax.ShapeDtypeStruct((_N, 3), jnp.float32),
    )(accp, b2)


def kernel(features, edge_index, W1, attn_l1, attn_r1, b1,
           W2, attn_l2, attn_r2, b2):
    src3 = edge_index[0].reshape(_NTILES, _NCH, _CHUNK)
    dst3 = edge_index[1].reshape(_NTILES, _NCH, _CHUNK)
    eye8 = jnp.eye(_H, dtype=jnp.float32)
    AL = (attn_l1[:, :, None] * eye8[:, None, :]).reshape(_H * _F, _H)
    AR = (attn_r1[:, :, None] * eye8[:, None, :]).reshape(_H * _F, _H)
    EXPAND = jnp.kron(jnp.fliplr(eye8), jnp.ones((1, _F), jnp.float32))
    va = (W2 @ attn_l2[0]).reshape(_H * _F, 1)
    vb = (W2 @ attn_r2[0]).reshape(_H * _F, 1)

    zext1, R1 = _tc1(features, W1, AL, AR)
    acc1 = _sc_edge_pass(src3, dst3, zext1, R1, _D1, True)
    zext2, R2 = _tc2(acc1, b1.reshape(1, _H * _F), W2, va, vb, EXPAND)
    acc2 = _sc_edge_pass(src3, dst3, zext2, R2, _D2, False)
    out = _tc3(acc2, b2.reshape(1, 3))
    return out.reshape(_N, 1, 3)


# speedup vs baseline: 123.4443x; 1.7085x over previous
import functools

import jax
import jax.numpy as jnp
from jax import lax
from jax.experimental import pallas as pl
from jax.experimental.pallas import tpu as pltpu
from jax.experimental.pallas import tpu_sc as plsc

_N = 10000
_E = 320000
_H = 8
_F = 16
_D1 = 144
_D2 = 16
_NC = 2
_NS = 16
_NTILES = _NC * _NS
_EPT = _E // _NTILES
_CHUNK = 50
_NCH = _EPT // _CHUNK
_CHUNK2 = 125
_RPS = 624
_RTAIL = _N - _NS * _RPS


def _tc1_body(x_ref, w_ref, al_ref, ar_ref, zext_ref, r_ref):
    z = jnp.dot(x_ref[...], w_ref[...], preferred_element_type=jnp.float32)
    el = jnp.dot(z, al_ref[...], preferred_element_type=jnp.float32)
    er = jnp.dot(z, ar_ref[...], preferred_element_type=jnp.float32)
    n = z.shape[0]
    zext_ref[...] = jnp.concatenate(
        [z, jnp.ones((n, _H), jnp.float32), el], axis=1)
    r_ref[...] = jnp.concatenate([er, er], axis=1)


def _tc1(x, W1, AL, AR):
    return pl.pallas_call(
        _tc1_body,
        out_shape=(
            jax.ShapeDtypeStruct((_N, _D1), jnp.float32),
            jax.ShapeDtypeStruct((_N, 16), jnp.float32),
        ),
    )(x, W1, AL, AR)


def _sc_edge_pass(src3, dst3, zext, rtab, d, multihead, chunk):
    nch = _EPT // chunk
    mesh = plsc.VectorSubcoreMesh(core_axis_name="c", subcore_axis_name="s")
    zeros = jnp.zeros((_N, d), jnp.float32)

    @functools.partial(
        pl.kernel,
        mesh=mesh,
        out_type=jax.ShapeDtypeStruct((_NC, _N, d), jnp.float32),
        compiler_params=pltpu.CompilerParams(use_tc_tiling_on_sc=False,
                                             needs_layout_passes=False),
        scratch_types=[
            pltpu.VMEM_SHARED((_N, d), jnp.float32),
            pltpu.VMEM((nch, chunk), jnp.int32),
            pltpu.VMEM((nch, chunk), jnp.int32),
            pltpu.VMEM((chunk, 16), jnp.float32),
            pltpu.VMEM((chunk, 16), jnp.float32),
            pltpu.VMEM((chunk, d), jnp.float32),
            pltpu.VMEM((chunk, d), jnp.float32),
            pltpu.SemaphoreType.DMA,
            pltpu.SemaphoreType.DMA,
            pltpu.SemaphoreType.DMA,
            pltpu.SemaphoreType.DMA,
        ],
    )
    def k(src_hbm, dst_hbm, z_hbm, r_hbm, zero_hbm, out_hbm,
          acc, srcv, dstv, rv0, rv1, zv0, zv1, sg0, sg1, ss0, ss1):
        cid = lax.axis_index("c")
        sid = lax.axis_index("s")
        wid = cid * _NS + sid
        rv = (rv0, rv1)
        zv = (zv0, zv1)
        sg = (sg0, sg1)
        ss = (ss0, ss1)

        pltpu.sync_copy(zero_hbm.at[pl.ds(sid * _RPS, _RPS)],
                        acc.at[pl.ds(sid * _RPS, _RPS)])

        @pl.when(sid == _NS - 1)
        def _zero_tail():
            pltpu.sync_copy(zero_hbm.at[pl.ds(_NS * _RPS, _RTAIL)],
                            acc.at[pl.ds(_NS * _RPS, _RTAIL)])

        pltpu.sync_copy(src_hbm.at[wid], srcv)
        pltpu.sync_copy(dst_hbm.at[wid], dstv)
        plsc.subcore_barrier()

        def issue_gather(b, g):
            pltpu.async_copy(r_hbm.at[dstv.at[g]], rv[b], sg[b])
            pltpu.async_copy(z_hbm.at[srcv.at[g]], zv[b], sg[b])

        def wait_gather(b, g):
            pltpu.make_async_copy(r_hbm.at[dstv.at[g]], rv[b], sg[b]).wait()
            pltpu.make_async_copy(z_hbm.at[srcv.at[g]], zv[b], sg[b]).wait()

        def issue_scatter(b, g):
            pltpu.async_copy(zv[b], acc.at[dstv.at[g]], ss[b], add=True)

        def wait_scatter(b, g):
            pltpu.make_async_copy(zv[b], acc.at[dstv.at[g]], ss[b]).wait()

        def compute(b):
            z = zv[b]
            r = rv[b]
            if multihead:
                @plsc.parallel_loop(0, chunk, unroll=2)
                def _msg(ci):
                    tail = z[ci, pl.ds(_H * _F, 16)]
                    e = tail + r[ci]
                    e = jnp.where(e >= 0.0, e, 0.2 * e)
                    w16 = jnp.exp(e)
                    for h in range(_H):
                        ws = w16[8 + h]
                        z[ci, pl.ds(h * _F, 16)] = ws * z[ci, pl.ds(h * _F, 16)]
                    z[ci, pl.ds(_H * _F, 16)] = lax.rev(w16, (0,)) * tail
            else:
                @plsc.parallel_loop(0, chunk, unroll=4)
                def _msg(ci):
                    zrow = z[ci]
                    bc = plsc.load_gather(
                        z, [jnp.full((16,), ci, jnp.int32),
                            jnp.full((16,), 4, jnp.int32)])
                    e = bc + r[ci]
                    e = jnp.where(e >= 0.0, e, 0.2 * e)
                    z[ci] = jnp.exp(e) * zrow

        issue_gather(0, 0)

        @pl.loop(0, nch, step=2)
        def _pair(g):
            issue_gather(1, g + 1)
            wait_gather(0, g)
            compute(0)
            issue_scatter(0, g)
            wait_gather(1, g + 1)
            compute(1)
            issue_scatter(1, g + 1)
            wait_scatter(0, g)

            @pl.when(g + 2 < nch)
            def _next():
                issue_gather(0, g + 2)

            wait_scatter(1, g + 1)

        plsc.subcore_barrier()
        pltpu.sync_copy(acc.at[pl.ds(sid * _RPS, _RPS)],
                        out_hbm.at[cid, pl.ds(sid * _RPS, _RPS)])

        @pl.when(sid == _NS - 1)
        def _out_tail():
            pltpu.sync_copy(acc.at[pl.ds(_NS * _RPS, _RTAIL)],
                            out_hbm.at[cid, pl.ds(_NS * _RPS, _RTAIL)])

    return k(src3, dst3, zext, rtab, zeros)


def _tc2_body(accp_ref, b1_ref, w2_ref, va_ref, vb_ref, expand_ref,
              zext_ref, r_ref):
    acc = accp_ref[0] + accp_ref[1]
    s = acc[:, _H * _F:_H * _F + _H]
    sx = jnp.dot(s, expand_ref[...], preferred_element_type=jnp.float32)
    h = acc[:, 0:_H * _F] / (sx + 1e-9) + b1_ref[...]
    h = jnp.where(h > 0.0, h, jnp.exp(h) - 1.0)
    z2 = jnp.dot(h, w2_ref[...], preferred_element_type=jnp.float32)
    el2 = jnp.dot(h, va_ref[...], preferred_element_type=jnp.float32)
    er2 = jnp.dot(h, vb_ref[...], preferred_element_type=jnp.float32)
    n = h.shape[0]
    one16 = jnp.ones((1, 16), jnp.float32)
    zext_ref[...] = jnp.concatenate(
        [z2, jnp.ones((n, 1), jnp.float32), el2,
         jnp.zeros((n, 11), jnp.float32)], axis=1)
    r_ref[...] = jnp.dot(er2, one16, preferred_element_type=jnp.float32)


def _tc2(accp, b1, W2, va, vb, EXPAND):
    return pl.pallas_call(
        _tc2_body,
        out_shape=(
            jax.ShapeDtypeStruct((_N, _D2), jnp.float32),
            jax.ShapeDtypeStruct((_N, 16), jnp.float32),
        ),
    )(accp, b1, W2, va, vb, EXPAND)


def _tc3_body(accp_ref, b2_ref, out_ref):
    acc = accp_ref[0] + accp_ref[1]
    sb = jnp.dot(acc[:, 3:4], jnp.ones((1, 16), jnp.float32),
                 preferred_element_type=jnp.float32)
    out_ref[...] = acc[:, 0:3] / (sb[:, 0:3] + 1e-9) + b2_ref[...]


def _tc3(accp, b2):
    return pl.pallas_call(
        _tc3_body,
        out_shape=jax.ShapeDtypeStruct((_N, 3), jnp.float32),
    )(accp, b2)


def kernel(features, edge_index, W1, attn_l1, attn_r1, b1,
           W2, attn_l2, attn_r2, b2):
    src3 = edge_index[0].reshape(_NTILES, _NCH, _CHUNK)
    dst3 = edge_index[1].reshape(_NTILES, _NCH, _CHUNK)
    eye8 = jnp.eye(_H, dtype=jnp.float32)
    AL = (attn_l1[:, :, None] * eye8[:, None, :]).reshape(_H * _F, _H)
    AR = (attn_r1[:, :, None] * eye8[:, None, :]).reshape(_H * _F, _H)
    EXPAND = jnp.kron(jnp.fliplr(eye8), jnp.ones((1, _F), jnp.float32))
    va = (W2 @ attn_l2[0]).reshape(_H * _F, 1)
    vb = (W2 @ attn_r2[0]).reshape(_H * _F, 1)

    zext1, R1 = _tc1(features, W1, AL, AR)
    acc1 = _sc_edge_pass(src3, dst3, zext1, R1, _D1, True, _CHUNK)
    src3b = edge_index[0].reshape(_NTILES, _EPT // _CHUNK2, _CHUNK2)
    dst3b = edge_index[1].reshape(_NTILES, _EPT // _CHUNK2, _CHUNK2)
    zext2, R2 = _tc2(acc1, b1.reshape(1, _H * _F), W2, va, vb, EXPAND)
    acc2 = _sc_edge_pass(src3b, dst3b, zext2, R2, _D2, False, _CHUNK2)
    out = _tc3(acc2, b2.reshape(1, 3))
    return out.reshape(_N, 1, 3)
